# baseline (speedup 1.0000x reference)
import functools

import jax
import jax.numpy as jnp
from jax import lax
from jax.experimental import pallas as pl
from jax.experimental.pallas import tpu as pltpu
from jax.experimental.pallas import tpu_sc as plsc

_G = 64
_W = 128
_CHR = 4
_NW = 32
_CB = 2048
_ZC = 2048


def _zero_vmem(buf, nwords):
    def body(i, _):
        buf[pl.ds(i * 16, 16)] = jnp.zeros((16,), jnp.float32)
        return 0
    lax.fori_loop(0, nwords // 16, body, 0)


def _zero_spmem(sid, zero_v, shared, nwords):
    for sl in range(nwords // _ZC):
        @pl.when(sid == (sl % 16))
        def _(sl=sl):
            pltpu.sync_copy(zero_v, shared.at[pl.ds(sl * _ZC, _ZC)])
    plsc.subcore_barrier()


def _fire_in(c, hbms, bufs, sem):
    r = pl.multiple_of(c * _CHR, _CHR)
    for h, b in zip(hbms, bufs):
        pltpu.async_copy(h.at[pl.ds(r, _CHR)], b, sem)


def _wait_in(hbms, bufs, sem):
    for h, b in zip(hbms, bufs):
        pltpu.make_async_copy(h.at[pl.ds(0, _CHR)], b, sem).wait()


def _fire_scatter(val_v, idx_v, shared, sem):
    for j in range(_CHR):
        pltpu.async_copy(val_v.at[j], shared.at[idx_v.at[j]], sem, add=True)


def _drain_scatter(val_v, idx_v, shared, sem):
    for j in range(_CHR):
        pltpu.make_async_copy(val_v.at[j], shared.at[idx_v.at[j]], sem).wait()


def _worker_split(total_ch, wid):
    base, rem = total_ch // _NW, total_ch % _NW
    nch_w = base + jnp.where(wid < rem, 1, 0)
    c_base = base * wid + jnp.minimum(wid, rem)
    return nch_w, c_base


def _make_deg_kernel(rows, n_pad):
    total_ch = rows // _CHR
    mesh = plsc.VectorSubcoreMesh(core_axis_name="c", subcore_axis_name="s")
    buf = pltpu.VMEM((_CHR, _W), jnp.float32)
    ibuf = pltpu.VMEM((_CHR, _W), jnp.int32)

    @functools.partial(
        pl.kernel,
        out_type=jax.ShapeDtypeStruct((2, n_pad), jnp.float32),
        mesh=mesh,
        compiler_params=pltpu.CompilerParams(needs_layout_passes=False),
        scratch_types=[
            ibuf, ibuf, buf, buf,
            pltpu.VMEM((_ZC,), jnp.float32),
            pltpu.VMEM_SHARED((n_pad,), jnp.float32),
            pltpu.SemaphoreType.DMA,
            pltpu.SemaphoreType.DMA,
            pltpu.SemaphoreType.DMA,
            pltpu.SemaphoreType.DMA,
        ],
    )
    def deg_kernel(dst_hbm, ew_hbm, out_hbm,
                   dst0, dst1, ew0, ew1, zero_v, deg_sh,
                   s_in0, s_in1, s_sc0, s_sc1):
        cid = lax.axis_index("c")
        sid = lax.axis_index("s")
        wid = sid * 2 + cid
        _zero_vmem(zero_v, _ZC)
        _zero_spmem(sid, zero_v, deg_sh, n_pad)
        nch_w, c_base = _worker_split(total_ch, wid)
        hbms = (dst_hbm, ew_hbm)
        _fire_in(c_base, hbms, (dst0, ew0), s_in0)

        def body2(k, _):
            c0 = c_base + 2 * k
            _wait_in(hbms, (dst0, ew0), s_in0)

            @pl.when(k > 0)
            def _():
                _drain_scatter(ew1, dst1, deg_sh, s_sc1)

            _fire_in(c0 + 1, hbms, (dst1, ew1), s_in1)
            _fire_scatter(ew0, dst0, deg_sh, s_sc0)

            _wait_in(hbms, (dst1, ew1), s_in1)
            _drain_scatter(ew0, dst0, deg_sh, s_sc0)

            @pl.when(2 * k + 2 < nch_w)
            def _():
                _fire_in(c0 + 2, hbms, (dst0, ew0), s_in0)

            _fire_scatter(ew1, dst1, deg_sh, s_sc1)
            return 0

        lax.fori_loop(0, nch_w // 2, body2, 0)

        @pl.when(nch_w % 2 == 1)
        def _():
            _wait_in(hbms, (dst0, ew0), s_in0)
            _fire_scatter(ew0, dst0, deg_sh, s_sc0)

        _drain_scatter(ew1, dst1, deg_sh, s_sc1)

        @pl.when(nch_w % 2 == 1)
        def _():
            _drain_scatter(ew0, dst0, deg_sh, s_sc0)

        plsc.subcore_barrier()

        @pl.when(sid == 0)
        def _():
            pltpu.sync_copy(deg_sh, out_hbm.at[cid])

    return deg_kernel


def _make_acc_kernel(rows, n_pad):
    total_ch = rows // _CHR
    mesh = plsc.VectorSubcoreMesh(core_axis_name="c", subcore_axis_name="s")
    buf = pltpu.VMEM((_CHR, _W), jnp.float32)
    ibuf = pltpu.VMEM((_CHR, _W), jnp.int32)

    @functools.partial(
        pl.kernel,
        out_type=jax.ShapeDtypeStruct((2, n_pad), jnp.float32),
        mesh=mesh,
        compiler_params=pltpu.CompilerParams(needs_layout_passes=False),
        scratch_types=[
            ibuf, ibuf, ibuf, ibuf, buf, buf, buf, buf,
            pltpu.VMEM((n_pad,), jnp.float32),
            pltpu.VMEM((_ZC,), jnp.float32),
            pltpu.VMEM_SHARED((n_pad,), jnp.float32),
            pltpu.SemaphoreType.DMA,
            pltpu.SemaphoreType.DMA,
            pltpu.SemaphoreType.DMA,
            pltpu.SemaphoreType.DMA,
        ],
    )
    def acc_kernel(src_hbm, dst_hbm, ew_hbm, p_hbm, out_hbm,
                   src0, src1, dst0, dst1, ew0, ew1, m0, m1,
                   p_v, zero_v, acc_sh, s_in0, s_in1, s_sc0, s_sc1):
        cid = lax.axis_index("c")
        sid = lax.axis_index("s")
        wid = sid * 2 + cid
        pltpu.sync_copy(p_hbm, p_v)
        _zero_vmem(zero_v, _ZC)
        _zero_spmem(sid, zero_v, acc_sh, n_pad)
        nch_w, c_base = _worker_split(total_ch, wid)
        hbms = (src_hbm, dst_hbm, ew_hbm)
        _fire_in(c_base, hbms, (src0, dst0, ew0), s_in0)

        def compute(src_v, ew_v, m_v):
            for j in range(_CHR):
                for t in range(_W // 16):
                    sl = pl.ds(t * 16, 16)
                    idx = src_v[j, sl]
                    pv = plsc.load_gather(p_v, [idx])
                    m_v[j, sl] = ew_v[j, sl] * pv

        def body2(k, _):
            c0 = c_base + 2 * k
            _wait_in(hbms, (src0, dst0, ew0), s_in0)
            compute(src0, ew0, m0)

            @pl.when(k > 0)
            def _():
                _drain_scatter(m1, dst1, acc_sh, s_sc1)

            _fire_in(c0 + 1, hbms, (src1, dst1, ew1), s_in1)
            _fire_scatter(m0, dst0, acc_sh, s_sc0)

            _wait_in(hbms, (src1, dst1, ew1), s_in1)
            compute(src1, ew1, m1)
            _drain_scatter(m0, dst0, acc_sh, s_sc0)

            @pl.when(2 * k + 2 < nch_w)
            def _():
                _fire_in(c0 + 2, hbms, (src0, dst0, ew0), s_in0)

            _fire_scatter(m1, dst1, acc_sh, s_sc1)
            return 0

        lax.fori_loop(0, nch_w // 2, body2, 0)

        @pl.when(nch_w % 2 == 1)
        def _():
            _wait_in(hbms, (src0, dst0, ew0), s_in0)
            compute(src0, ew0, m0)
            _fire_scatter(m0, dst0, acc_sh, s_sc0)

        _drain_scatter(m1, dst1, acc_sh, s_sc1)

        @pl.when(nch_w % 2 == 1)
        def _():
            _drain_scatter(m0, dst0, acc_sh, s_sc0)

        plsc.subcore_barrier()

        @pl.when(sid == 0)
        def _():
            pltpu.sync_copy(acc_sh, out_hbm.at[cid])

    return acc_kernel


def _node_stage(xT, v, d0, d1, nb, f):
    def body(xT_ref, v_ref, d0_ref, d1_ref, p_ref, q_ref, dis_ref):
        z = jnp.sum(xT_ref[...] * v_ref[...], axis=0, keepdims=True)
        deg = d0_ref[...] + d1_ref[...] + 1.0
        dis = jnp.where(deg > 0, lax.rsqrt(deg), 0.0)
        p_ref[...] = dis * z
        q_ref[...] = z / deg
        dis_ref[...] = dis

    vec = jax.ShapeDtypeStruct((1, nb * _CB), jnp.float32)
    vspec = pl.BlockSpec((1, _CB), lambda i: (0, i))
    return pl.pallas_call(
        body,
        grid=(nb,),
        in_specs=[
            pl.BlockSpec((f, _CB), lambda i: (0, i)),
            pl.BlockSpec((f, 1), lambda i: (0, 0)),
            vspec,
            vspec,
        ],
        out_specs=[vspec, vspec, vspec],
        out_shape=[vec, vec, vec],
    )(xT, v, d0, d1)


def _pool_stage(a0, a1, dis, q, bat, params, nb):
    def body(a0_ref, a1_ref, dis_ref, q_ref, b_ref, prm_ref, out_ref, cnt_ref):
        i = pl.program_id(0)

        @pl.when(i == 0)
        def _():
            out_ref[...] = jnp.zeros_like(out_ref)
            cnt_ref[...] = jnp.zeros_like(cnt_ref)

        y = dis_ref[...] * (a0_ref[...] + a1_ref[...]) + q_ref[...]
        gids = lax.broadcasted_iota(jnp.int32, (_G, _CB), 0)
        m = (b_ref[...] == gids).astype(jnp.float32)
        out_ref[...] += jnp.sum(m * y, axis=1, keepdims=True)
        cnt_ref[...] += jnp.sum(m, axis=1, keepdims=True)

        @pl.when(i == nb - 1)
        def _():
            cnt = cnt_ref[...]
            c1 = prm_ref[0, 0]
            blv = prm_ref[0, 1]
            out_ref[...] = ((out_ref[...] + cnt * c1)
                            / jnp.maximum(cnt, 1.0) + blv)

    vspec = pl.BlockSpec((1, _CB), lambda i: (0, i))
    return pl.pallas_call(
        body,
        grid=(nb,),
        in_specs=[
            vspec,
            vspec,
            vspec,
            vspec,
            vspec,
            pl.BlockSpec((1, 2), lambda i: (0, 0)),
        ],
        out_specs=pl.BlockSpec((_G, 1), lambda i: (0, 0)),
        out_shape=jax.ShapeDtypeStruct((_G, 1), jnp.float32),
        scratch_shapes=[pltpu.VMEM((_G, 1), jnp.float32)],
    )(a0, a1, dis, q, bat, params)


def kernel(x, edge_index, edge_attr, batch, W1, b1, W2, b2, W3, b3, Wl, bl):
    n, f = x.shape
    e = edge_attr.shape[0]

    epc = _CHR * _W
    e_pad = (-(-e // epc)) * epc
    rows = e_pad // _W
    nb = -(-n // _CB)
    n_pad = nb * _CB

    src = jnp.pad(edge_index[0], (0, e_pad - e)).reshape(rows, _W)
    dst = jnp.pad(edge_index[1], (0, e_pad - e)).reshape(rows, _W)
    ew = jnp.pad(edge_attr, (0, e_pad - e)).reshape(rows, _W)
    bat = jnp.pad(batch, (0, n_pad - n), constant_values=_G).reshape(1, n_pad)
    xT = jnp.pad(x, ((0, n_pad - n), (0, 0))).T
    v = (W3 @ Wl).reshape(f, 1)
    params = jnp.concatenate([b3 @ Wl, bl]).reshape(1, 2)

    deg2 = _make_deg_kernel(rows, n_pad)(dst, ew)
    d0 = deg2[0].reshape(1, n_pad)
    d1 = deg2[1].reshape(1, n_pad)
    p, q, dis = _node_stage(xT, v, d0, d1, nb, f)

    acc2 = _make_acc_kernel(rows, n_pad)(src, dst, ew, p.reshape(-1))
    a0 = acc2[0].reshape(1, n_pad)
    a1 = acc2[1].reshape(1, n_pad)
    return _pool_stage(a0, a1, dis, q, bat, params, nb)


# speedup vs baseline: 1.4353x; 1.4353x over previous
import functools

import jax
import jax.numpy as jnp
from jax import lax
from jax.experimental import pallas as pl
from jax.experimental.pallas import tpu as pltpu
from jax.experimental.pallas import tpu_sc as plsc

_G = 64
_W = 128
_CHR = 8
_NW = 32
_CB = 2048
_ZC = 2048


def _zero_vmem(buf, nwords):
    def body(i, _):
        buf[pl.ds(i * 16, 16)] = jnp.zeros((16,), jnp.float32)
        return 0
    lax.fori_loop(0, nwords // 16, body, 0)


def _zero_spmem(sid, zero_v, shared, nwords):
    for sl in range(nwords // _ZC):
        @pl.when(sid == (sl % 16))
        def _(sl=sl):
            pltpu.sync_copy(zero_v, shared.at[pl.ds(sl * _ZC, _ZC)])
    plsc.subcore_barrier()


def _fire_in(c, hbms, bufs, sem):
    r = pl.multiple_of(c * _CHR, _CHR)
    for h, b in zip(hbms, bufs):
        pltpu.async_copy(h.at[pl.ds(r, _CHR)], b, sem)


def _wait_in(hbms, bufs, sem):
    for h, b in zip(hbms, bufs):
        pltpu.make_async_copy(h.at[pl.ds(0, _CHR)], b, sem).wait()


def _fire_scatter(val_v, idx_v, shared, sem):
    for j in range(_CHR):
        pltpu.async_copy(val_v.at[j], shared.at[idx_v.at[j]], sem, add=True)


def _drain_scatter(val_v, idx_v, shared, sem):
    for j in range(_CHR):
        pltpu.make_async_copy(val_v.at[j], shared.at[idx_v.at[j]], sem).wait()


def _worker_split(total_ch, wid):
    base, rem = total_ch // _NW, total_ch % _NW
    nch_w = base + jnp.where(wid < rem, 1, 0)
    c_base = base * wid + jnp.minimum(wid, rem)
    return nch_w, c_base


def _make_deg_kernel(rows, n_pad):
    total_ch = rows // _CHR
    mesh = plsc.VectorSubcoreMesh(core_axis_name="c", subcore_axis_name="s")
    buf = pltpu.VMEM((_CHR, _W), jnp.float32)
    ibuf = pltpu.VMEM((_CHR, _W), jnp.int32)

    @functools.partial(
        pl.kernel,
        out_type=jax.ShapeDtypeStruct((2, n_pad), jnp.float32),
        mesh=mesh,
        compiler_params=pltpu.CompilerParams(needs_layout_passes=False),
        scratch_types=[
            ibuf, ibuf, buf, buf,
            pltpu.VMEM((_ZC,), jnp.float32),
            pltpu.VMEM_SHARED((n_pad,), jnp.float32),
            pltpu.SemaphoreType.DMA,
            pltpu.SemaphoreType.DMA,
            pltpu.SemaphoreType.DMA,
            pltpu.SemaphoreType.DMA,
        ],
    )
    def deg_kernel(dst_hbm, ew_hbm, out_hbm,
                   dst0, dst1, ew0, ew1, zero_v, deg_sh,
                   s_in0, s_in1, s_sc0, s_sc1):
        cid = lax.axis_index("c")
        sid = lax.axis_index("s")
        wid = sid * 2 + cid
        _zero_vmem(zero_v, _ZC)
        _zero_spmem(sid, zero_v, deg_sh, n_pad)
        nch_w, c_base = _worker_split(total_ch, wid)
        hbms = (dst_hbm, ew_hbm)
        _fire_in(c_base, hbms, (dst0, ew0), s_in0)

        def body2(k, _):
            c0 = c_base + 2 * k
            _wait_in(hbms, (dst0, ew0), s_in0)

            @pl.when(k > 0)
            def _():
                _drain_scatter(ew1, dst1, deg_sh, s_sc1)

            _fire_in(c0 + 1, hbms, (dst1, ew1), s_in1)
            _fire_scatter(ew0, dst0, deg_sh, s_sc0)

            _wait_in(hbms, (dst1, ew1), s_in1)
            _drain_scatter(ew0, dst0, deg_sh, s_sc0)

            @pl.when(2 * k + 2 < nch_w)
            def _():
                _fire_in(c0 + 2, hbms, (dst0, ew0), s_in0)

            _fire_scatter(ew1, dst1, deg_sh, s_sc1)
            return 0

        lax.fori_loop(0, nch_w // 2, body2, 0)

        @pl.when(nch_w % 2 == 1)
        def _():
            _wait_in(hbms, (dst0, ew0), s_in0)
            _fire_scatter(ew0, dst0, deg_sh, s_sc0)

        _drain_scatter(ew1, dst1, deg_sh, s_sc1)

        @pl.when(nch_w % 2 == 1)
        def _():
            _drain_scatter(ew0, dst0, deg_sh, s_sc0)

        plsc.subcore_barrier()

        @pl.when(sid == 0)
        def _():
            pltpu.sync_copy(deg_sh, out_hbm.at[cid])

    return deg_kernel


def _make_acc_kernel(rows, n_pad):
    total_ch = rows // _CHR
    mesh = plsc.VectorSubcoreMesh(core_axis_name="c", subcore_axis_name="s")
    buf = pltpu.VMEM((_CHR, _W), jnp.float32)
    ibuf = pltpu.VMEM((_CHR, _W), jnp.int32)

    @functools.partial(
        pl.kernel,
        out_type=jax.ShapeDtypeStruct((2, n_pad), jnp.float32),
        mesh=mesh,
        compiler_params=pltpu.CompilerParams(needs_layout_passes=False),
        scratch_types=[
            ibuf, ibuf, ibuf, ibuf, buf, buf, buf, buf,
            pltpu.VMEM((n_pad,), jnp.float32),
            pltpu.VMEM((_ZC,), jnp.float32),
            pltpu.VMEM_SHARED((n_pad,), jnp.float32),
            pltpu.SemaphoreType.DMA,
            pltpu.SemaphoreType.DMA,
            pltpu.SemaphoreType.DMA,
            pltpu.SemaphoreType.DMA,
        ],
    )
    def acc_kernel(src_hbm, dst_hbm, ew_hbm, p_hbm, out_hbm,
                   src0, src1, dst0, dst1, ew0, ew1, m0, m1,
                   p_v, zero_v, acc_sh, s_in0, s_in1, s_sc0, s_sc1):
        cid = lax.axis_index("c")
        sid = lax.axis_index("s")
        wid = sid * 2 + cid
        pltpu.sync_copy(p_hbm, p_v)
        _zero_vmem(zero_v, _ZC)
        _zero_spmem(sid, zero_v, acc_sh, n_pad)
        nch_w, c_base = _worker_split(total_ch, wid)
        hbms = (src_hbm, dst_hbm, ew_hbm)
        _fire_in(c_base, hbms, (src0, dst0, ew0), s_in0)

        ngrp = _W // 16

        def compute(src_v, ew_v, m_v):
            @plsc.parallel_loop(0, _CHR * ngrp, unroll=4)
            def _(g):
                j = g // ngrp
                sl = pl.ds(pl.multiple_of((g % ngrp) * 16, 16), 16)
                idx = src_v[j, sl]
                pv = plsc.load_gather(p_v, [idx])
                m_v[j, sl] = ew_v[j, sl] * pv

        def body2(k, _):
            c0 = c_base + 2 * k
            _wait_in(hbms, (src0, dst0, ew0), s_in0)
            compute(src0, ew0, m0)

            @pl.when(k > 0)
            def _():
                _drain_scatter(m1, dst1, acc_sh, s_sc1)

            _fire_in(c0 + 1, hbms, (src1, dst1, ew1), s_in1)
            _fire_scatter(m0, dst0, acc_sh, s_sc0)

            _wait_in(hbms, (src1, dst1, ew1), s_in1)
            compute(src1, ew1, m1)
            _drain_scatter(m0, dst0, acc_sh, s_sc0)

            @pl.when(2 * k + 2 < nch_w)
            def _():
                _fire_in(c0 + 2, hbms, (src0, dst0, ew0), s_in0)

            _fire_scatter(m1, dst1, acc_sh, s_sc1)
            return 0

        lax.fori_loop(0, nch_w // 2, body2, 0)

        @pl.when(nch_w % 2 == 1)
        def _():
            _wait_in(hbms, (src0, dst0, ew0), s_in0)
            compute(src0, ew0, m0)
            _fire_scatter(m0, dst0, acc_sh, s_sc0)

        _drain_scatter(m1, dst1, acc_sh, s_sc1)

        @pl.when(nch_w % 2 == 1)
        def _():
            _drain_scatter(m0, dst0, acc_sh, s_sc0)

        plsc.subcore_barrier()

        @pl.when(sid == 0)
        def _():
            pltpu.sync_copy(acc_sh, out_hbm.at[cid])

    return acc_kernel


def _node_stage(xT, v, d0, d1, nb, f):
    def body(xT_ref, v_ref, d0_ref, d1_ref, p_ref, q_ref, dis_ref):
        z = jnp.sum(xT_ref[...] * v_ref[...], axis=0, keepdims=True)
        deg = d0_ref[...] + d1_ref[...] + 1.0
        dis = jnp.where(deg > 0, lax.rsqrt(deg), 0.0)
        p_ref[...] = dis * z
        q_ref[...] = z / deg
        dis_ref[...] = dis

    vec = jax.ShapeDtypeStruct((1, nb * _CB), jnp.float32)
    vspec = pl.BlockSpec((1, _CB), lambda i: (0, i))
    return pl.pallas_call(
        body,
        grid=(nb,),
        in_specs=[
            pl.BlockSpec((f, _CB), lambda i: (0, i)),
            pl.BlockSpec((f, 1), lambda i: (0, 0)),
            vspec,
            vspec,
        ],
        out_specs=[vspec, vspec, vspec],
        out_shape=[vec, vec, vec],
    )(xT, v, d0, d1)


def _pool_stage(a0, a1, dis, q, bat, params, nb):
    def body(a0_ref, a1_ref, dis_ref, q_ref, b_ref, prm_ref, out_ref, cnt_ref):
        i = pl.program_id(0)

        @pl.when(i == 0)
        def _():
            out_ref[...] = jnp.zeros_like(out_ref)
            cnt_ref[...] = jnp.zeros_like(cnt_ref)

        y = dis_ref[...] * (a0_ref[...] + a1_ref[...]) + q_ref[...]
        gids = lax.broadcasted_iota(jnp.int32, (_G, _CB), 0)
        m = (b_ref[...] == gids).astype(jnp.float32)
        out_ref[...] += jnp.sum(m * y, axis=1, keepdims=True)
        cnt_ref[...] += jnp.sum(m, axis=1, keepdims=True)

        @pl.when(i == nb - 1)
        def _():
            cnt = cnt_ref[...]
            c1 = prm_ref[0, 0]
            blv = prm_ref[0, 1]
            out_ref[...] = ((out_ref[...] + cnt * c1)
                            / jnp.maximum(cnt, 1.0) + blv)

    vspec = pl.BlockSpec((1, _CB), lambda i: (0, i))
    return pl.pallas_call(
        body,
        grid=(nb,),
        in_specs=[
            vspec,
            vspec,
            vspec,
            vspec,
            vspec,
            pl.BlockSpec((1, 2), lambda i: (0, 0)),
        ],
        out_specs=pl.BlockSpec((_G, 1), lambda i: (0, 0)),
        out_shape=jax.ShapeDtypeStruct((_G, 1), jnp.float32),
        scratch_shapes=[pltpu.VMEM((_G, 1), jnp.float32)],
    )(a0, a1, dis, q, bat, params)


def kernel(x, edge_index, edge_attr, batch, W1, b1, W2, b2, W3, b3, Wl, bl):
    n, f = x.shape
    e = edge_attr.shape[0]

    epc = _CHR * _W
    e_pad = (-(-e // epc)) * epc
    rows = e_pad // _W
    nb = -(-n // _CB)
    n_pad = nb * _CB

    src = jnp.pad(edge_index[0], (0, e_pad - e)).reshape(rows, _W)
    dst = jnp.pad(edge_index[1], (0, e_pad - e)).reshape(rows, _W)
    ew = jnp.pad(edge_attr, (0, e_pad - e)).reshape(rows, _W)
    bat = jnp.pad(batch, (0, n_pad - n), constant_values=_G).reshape(1, n_pad)
    xT = jnp.pad(x, ((0, n_pad - n), (0, 0))).T
    v = (W3 @ Wl).reshape(f, 1)
    params = jnp.concatenate([b3 @ Wl, bl]).reshape(1, 2)

    deg2 = _make_deg_kernel(rows, n_pad)(dst, ew)
    d0 = deg2[0].reshape(1, n_pad)
    d1 = deg2[1].reshape(1, n_pad)
    p, q, dis = _node_stage(xT, v, d0, d1, nb, f)

    acc2 = _make_acc_kernel(rows, n_pad)(src, dst, ew, p.reshape(-1))
    a0 = acc2[0].reshape(1, n_pad)
    a1 = acc2[1].reshape(1, n_pad)
    return _pool_stage(a0, a1, dis, q, bat, params, nb)


# speedup vs baseline: 1.4400x; 1.0033x over previous
import functools

import jax
import jax.numpy as jnp
from jax import lax
from jax.experimental import pallas as pl
from jax.experimental.pallas import tpu as pltpu
from jax.experimental.pallas import tpu_sc as plsc

_G = 64
_W = 128
_CHR = 8
_NW = 32
_CB = 2048
_ZC = 2048


def _zero_vmem(buf, nwords):
    def body(i, _):
        buf[pl.ds(i * 16, 16)] = jnp.zeros((16,), jnp.float32)
        return 0
    lax.fori_loop(0, nwords // 16, body, 0)


def _zero_spmem(sid, zero_v, shared, nwords):
    for sl in range(nwords // _ZC):
        @pl.when(sid == (sl % 16))
        def _(sl=sl):
            pltpu.sync_copy(zero_v, shared.at[pl.ds(sl * _ZC, _ZC)])
    plsc.subcore_barrier()


def _fire_in(c, hbms, bufs, sem):
    r = pl.multiple_of(c * _CHR, _CHR)
    for h, b in zip(hbms, bufs):
        pltpu.async_copy(h.at[pl.ds(r, _CHR)], b, sem)


def _wait_in(hbms, bufs, sem):
    for h, b in zip(hbms, bufs):
        pltpu.make_async_copy(h.at[pl.ds(0, _CHR)], b, sem).wait()


def _fire_scatter(val_v, idx_v, shared, sem):
    for j in range(_CHR):
        pltpu.async_copy(val_v.at[j], shared.at[idx_v.at[j]], sem, add=True)


def _drain_scatter(val_v, idx_v, shared, sem):
    for j in range(_CHR):
        pltpu.make_async_copy(val_v.at[j], shared.at[idx_v.at[j]], sem).wait()


def _worker_split(total_ch, wid):
    base, rem = total_ch // _NW, total_ch % _NW
    nch_w = base + jnp.where(wid < rem, 1, 0)
    c_base = base * wid + jnp.minimum(wid, rem)
    return nch_w, c_base


def _make_deg_kernel(rows, n_pad):
    total_ch = rows // _CHR
    mesh = plsc.VectorSubcoreMesh(core_axis_name="c", subcore_axis_name="s")
    buf = pltpu.VMEM((_CHR, _W), jnp.float32)
    ibuf = pltpu.VMEM((_CHR, _W), jnp.int32)

    @functools.partial(
        pl.kernel,
        out_type=jax.ShapeDtypeStruct((2, n_pad), jnp.float32),
        mesh=mesh,
        compiler_params=pltpu.CompilerParams(needs_layout_passes=False),
        scratch_types=[
            ibuf, ibuf, buf, buf,
            pltpu.VMEM((_ZC,), jnp.float32),
            pltpu.VMEM_SHARED((n_pad,), jnp.float32),
            pltpu.SemaphoreType.DMA,
            pltpu.SemaphoreType.DMA,
            pltpu.SemaphoreType.DMA,
            pltpu.SemaphoreType.DMA,
        ],
    )
    def deg_kernel(dst_hbm, ew_hbm, out_hbm,
                   dst0, dst1, ew0, ew1, zero_v, deg_sh,
                   s_in0, s_in1, s_sc0, s_sc1):
        cid = lax.axis_index("c")
        sid = lax.axis_index("s")
        wid = sid * 2 + cid
        _zero_vmem(zero_v, _ZC)
        _zero_spmem(sid, zero_v, deg_sh, n_pad)
        nch_w, c_base = _worker_split(total_ch, wid)
        hbms = (dst_hbm, ew_hbm)
        _fire_in(c_base, hbms, (dst0, ew0), s_in0)

        def body2(k, _):
            c0 = c_base + 2 * k
            _wait_in(hbms, (dst0, ew0), s_in0)

            @pl.when(k > 0)
            def _():
                _drain_scatter(ew1, dst1, deg_sh, s_sc1)

            _fire_in(c0 + 1, hbms, (dst1, ew1), s_in1)
            _fire_scatter(ew0, dst0, deg_sh, s_sc0)

            _wait_in(hbms, (dst1, ew1), s_in1)
            _drain_scatter(ew0, dst0, deg_sh, s_sc0)

            @pl.when(2 * k + 2 < nch_w)
            def _():
                _fire_in(c0 + 2, hbms, (dst0, ew0), s_in0)

            _fire_scatter(ew1, dst1, deg_sh, s_sc1)
            return 0

        lax.fori_loop(0, nch_w // 2, body2, 0)

        @pl.when(nch_w % 2 == 1)
        def _():
            _wait_in(hbms, (dst0, ew0), s_in0)
            _fire_scatter(ew0, dst0, deg_sh, s_sc0)

        _drain_scatter(ew1, dst1, deg_sh, s_sc1)

        @pl.when(nch_w % 2 == 1)
        def _():
            _drain_scatter(ew0, dst0, deg_sh, s_sc0)

        plsc.subcore_barrier()

        @pl.when(sid == 0)
        def _():
            pltpu.sync_copy(deg_sh, out_hbm.at[cid])

    return deg_kernel


def _make_acc_kernel(rows, n_pad):
    total_ch = rows // _CHR
    mesh = plsc.VectorSubcoreMesh(core_axis_name="c", subcore_axis_name="s")
    buf = pltpu.VMEM((_CHR, _W), jnp.float32)
    ibuf = pltpu.VMEM((_CHR, _W), jnp.int32)

    @functools.partial(
        pl.kernel,
        out_type=jax.ShapeDtypeStruct((2, n_pad), jnp.float32),
        mesh=mesh,
        compiler_params=pltpu.CompilerParams(needs_layout_passes=False),
        scratch_types=[
            ibuf, ibuf, ibuf, ibuf, buf, buf, buf, buf,
            pltpu.VMEM((n_pad,), jnp.float32),
            pltpu.VMEM((_ZC,), jnp.float32),
            pltpu.VMEM_SHARED((n_pad,), jnp.float32),
            pltpu.SemaphoreType.DMA,
            pltpu.SemaphoreType.DMA,
            pltpu.SemaphoreType.DMA,
            pltpu.SemaphoreType.DMA,
        ],
    )
    def acc_kernel(src_hbm, dst_hbm, ew_hbm, p_hbm, out_hbm,
                   src0, src1, dst0, dst1, ew0, ew1, m0, m1,
                   p_v, zero_v, acc_sh, s_in0, s_in1, s_sc0, s_sc1):
        cid = lax.axis_index("c")
        sid = lax.axis_index("s")
        wid = sid * 2 + cid
        pltpu.sync_copy(p_hbm, p_v)
        _zero_vmem(zero_v, _ZC)
        _zero_spmem(sid, zero_v, acc_sh, n_pad)
        nch_w, c_base = _worker_split(total_ch, wid)
        hbms = (src_hbm, dst_hbm, ew_hbm)
        _fire_in(c_base, hbms, (src0, dst0, ew0), s_in0)

        ngrp = _W // 16

        def compute(src_v, ew_v, m_v):
            @plsc.parallel_loop(0, _CHR * ngrp, unroll=8)
            def _(g):
                j = g // ngrp
                sl = pl.ds(pl.multiple_of((g % ngrp) * 16, 16), 16)
                idx = src_v[j, sl]
                pv = plsc.load_gather(p_v, [idx])
                m_v[j, sl] = ew_v[j, sl] * pv

        def body2(k, _):
            c0 = c_base + 2 * k
            _wait_in(hbms, (src0, dst0, ew0), s_in0)
            compute(src0, ew0, m0)

            @pl.when(k > 0)
            def _():
                _drain_scatter(m1, dst1, acc_sh, s_sc1)

            _fire_in(c0 + 1, hbms, (src1, dst1, ew1), s_in1)
            _fire_scatter(m0, dst0, acc_sh, s_sc0)

            _wait_in(hbms, (src1, dst1, ew1), s_in1)
            compute(src1, ew1, m1)
            _drain_scatter(m0, dst0, acc_sh, s_sc0)

            @pl.when(2 * k + 2 < nch_w)
            def _():
                _fire_in(c0 + 2, hbms, (src0, dst0, ew0), s_in0)

            _fire_scatter(m1, dst1, acc_sh, s_sc1)
            return 0

        lax.fori_loop(0, nch_w // 2, body2, 0)

        @pl.when(nch_w % 2 == 1)
        def _():
            _wait_in(hbms, (src0, dst0, ew0), s_in0)
            compute(src0, ew0, m0)
            _fire_scatter(m0, dst0, acc_sh, s_sc0)

        _drain_scatter(m1, dst1, acc_sh, s_sc1)

        @pl.when(nch_w % 2 == 1)
        def _():
            _drain_scatter(m0, dst0, acc_sh, s_sc0)

        plsc.subcore_barrier()

        @pl.when(sid == 0)
        def _():
            pltpu.sync_copy(acc_sh, out_hbm.at[cid])

    return acc_kernel


def _node_stage(xT, v, d0, d1, nb, f):
    def body(xT_ref, v_ref, d0_ref, d1_ref, p_ref, q_ref, dis_ref):
        z = jnp.sum(xT_ref[...] * v_ref[...], axis=0, keepdims=True)
        deg = d0_ref[...] + d1_ref[...] + 1.0
        dis = jnp.where(deg > 0, lax.rsqrt(deg), 0.0)
        p_ref[...] = dis * z
        q_ref[...] = z / deg
        dis_ref[...] = dis

    vec = jax.ShapeDtypeStruct((1, nb * _CB), jnp.float32)
    vspec = pl.BlockSpec((1, _CB), lambda i: (0, i))
    return pl.pallas_call(
        body,
        grid=(nb,),
        in_specs=[
            pl.BlockSpec((f, _CB), lambda i: (0, i)),
            pl.BlockSpec((f, 1), lambda i: (0, 0)),
            vspec,
            vspec,
        ],
        out_specs=[vspec, vspec, vspec],
        out_shape=[vec, vec, vec],
    )(xT, v, d0, d1)


def _pool_stage(a0, a1, dis, q, bat, params, nb):
    def body(a0_ref, a1_ref, dis_ref, q_ref, b_ref, prm_ref, out_ref, cnt_ref):
        i = pl.program_id(0)

        @pl.when(i == 0)
        def _():
            out_ref[...] = jnp.zeros_like(out_ref)
            cnt_ref[...] = jnp.zeros_like(cnt_ref)

        y = dis_ref[...] * (a0_ref[...] + a1_ref[...]) + q_ref[...]
        gids = lax.broadcasted_iota(jnp.int32, (_G, _CB), 0)
        m = (b_ref[...] == gids).astype(jnp.float32)
        out_ref[...] += jnp.sum(m * y, axis=1, keepdims=True)
        cnt_ref[...] += jnp.sum(m, axis=1, keepdims=True)

        @pl.when(i == nb - 1)
        def _():
            cnt = cnt_ref[...]
            c1 = prm_ref[0, 0]
            blv = prm_ref[0, 1]
            out_ref[...] = ((out_ref[...] + cnt * c1)
                            / jnp.maximum(cnt, 1.0) + blv)

    vspec = pl.BlockSpec((1, _CB), lambda i: (0, i))
    return pl.pallas_call(
        body,
        grid=(nb,),
        in_specs=[
            vspec,
            vspec,
            vspec,
            vspec,
            vspec,
            pl.BlockSpec((1, 2), lambda i: (0, 0)),
        ],
        out_specs=pl.BlockSpec((_G, 1), lambda i: (0, 0)),
        out_shape=jax.ShapeDtypeStruct((_G, 1), jnp.float32),
        scratch_shapes=[pltpu.VMEM((_G, 1), jnp.float32)],
    )(a0, a1, dis, q, bat, params)


def kernel(x, edge_index, edge_attr, batch, W1, b1, W2, b2, W3, b3, Wl, bl):
    n, f = x.shape
    e = edge_attr.shape[0]

    epc = _CHR * _W
    e_pad = (-(-e // epc)) * epc
    rows = e_pad // _W
    nb = -(-n // _CB)
    n_pad = nb * _CB

    src = jnp.pad(edge_index[0], (0, e_pad - e)).reshape(rows, _W)
    dst = jnp.pad(edge_index[1], (0, e_pad - e)).reshape(rows, _W)
    ew = jnp.pad(edge_attr, (0, e_pad - e)).reshape(rows, _W)
    bat = jnp.pad(batch, (0, n_pad - n), constant_values=_G).reshape(1, n_pad)
    xT = jnp.pad(x, ((0, n_pad - n), (0, 0))).T
    v = (W3 @ Wl).reshape(f, 1)
    params = jnp.concatenate([b3 @ Wl, bl]).reshape(1, 2)

    deg2 = _make_deg_kernel(rows, n_pad)(dst, ew)
    d0 = deg2[0].reshape(1, n_pad)
    d1 = deg2[1].reshape(1, n_pad)
    p, q, dis = _node_stage(xT, v, d0, d1, nb, f)

    acc2 = _make_acc_kernel(rows, n_pad)(src, dst, ew, p.reshape(-1))
    a0 = acc2[0].reshape(1, n_pad)
    a1 = acc2[1].reshape(1, n_pad)
    return _pool_stage(a0, a1, dis, q, bat, params, nb)
